# Initial kernel scaffold; baseline (speedup 1.0000x reference)
import jax
import jax.numpy as jnp
from jax.experimental import pallas as pl


def kernel(x, W0, W1, W2, W3, W4, W5, W6, W7, W8, W9):
    raise NotImplementedError("write your pallas kernel here")


# speedup vs baseline: 108.8838x; 108.8838x over previous
import functools

import jax
import jax.numpy as jnp
from jax import lax
from jax.experimental import pallas as pl
from jax.experimental.pallas import tpu as pltpu
from jax.experimental.pallas import tpu_sc as plsc

BS, NV, SEQ = 4096, 26, 200
ROW = NV * SEQ
NCAT = 10
GW = NCAT * SEQ
VOCAB = 1000
LANES = 16
VPR = GW // LANES
NC, NS = 2, 16
NWORKERS = NC * NS
ROWS_PER_TILE = BS // NWORKERS


def _chan_offset(v: int) -> jax.Array:
    base = v * LANES
    c0, c1 = base // SEQ, (base + LANES - 1) // SEQ
    if c0 == c1:
        return jnp.full((LANES,), c0 * VOCAB, jnp.int32)
    k = c1 * SEQ - base
    return jnp.where(lax.iota(jnp.int32, LANES) < k,
                     jnp.int32(c0 * VOCAB), jnp.int32(c1 * VOCAB))


def _sc_body(x_hbm, tab_hbm, out_hbm, tab_v, buf_v, sem):
    wid = lax.axis_index("s") * NC + lax.axis_index("c")
    pltpu.sync_copy(tab_hbm, tab_v)
    row0 = wid * ROWS_PER_TILE

    def do_row(i, carry):
        off = (row0 + i) * ROW
        off = pl.multiple_of(off, 8)
        pltpu.async_copy(x_hbm.at[pl.ds(off, ROW)], buf_v, sem).wait()
        for v in range(VPR):
            o = v * LANES
            idx = buf_v[pl.ds(o, LANES)].astype(jnp.int32) + _chan_offset(v)
            buf_v[pl.ds(o, LANES)] = plsc.load_gather(tab_v, [idx])
        pltpu.async_copy(buf_v, out_hbm.at[pl.ds(off, ROW)], sem).wait()
        return carry

    lax.fori_loop(0, ROWS_PER_TILE, do_row, 0)


@functools.partial(jax.jit, static_argnames=())
def _run(x_flat, tab):
    mesh = plsc.VectorSubcoreMesh(core_axis_name="c", subcore_axis_name="s")
    return pl.kernel(
        _sc_body,
        out_type=jax.ShapeDtypeStruct((BS * ROW,), jnp.float32),
        mesh=mesh,
        scratch_types=[
            pltpu.VMEM((NCAT * VOCAB,), jnp.float32),
            pltpu.VMEM((ROW,), jnp.float32),
            pltpu.SemaphoreType.DMA,
        ],
        compiler_params=pltpu.CompilerParams(needs_layout_passes=False),
    )(x_flat, tab)


def kernel(x, W0, W1, W2, W3, W4, W5, W6, W7, W8, W9):
    tab = jnp.concatenate(
        [W0, W1, W2, W3, W4, W5, W6, W7, W8, W9], axis=0
    ).reshape(NCAT * VOCAB)
    out = _run(x.reshape(BS * ROW), tab)
    return out.reshape(BS, NV, SEQ)


# speedup vs baseline: 126.6015x; 1.1627x over previous
import functools

import jax
import jax.numpy as jnp
from jax import lax
from jax.experimental import pallas as pl
from jax.experimental.pallas import tpu as pltpu
from jax.experimental.pallas import tpu_sc as plsc

BS, NV, SEQ = 4096, 26, 200
ROW = NV * SEQ
NCAT = 10
GW = NCAT * SEQ
VOCAB = 1000
LANES = 16
VPR = GW // LANES
NC, NS = 2, 16
NWORKERS = NC * NS
ROWS_PER_TILE = BS // NWORKERS
NB = 4
CW = NB * ROW
NCHUNK = ROWS_PER_TILE // NB
NBUF = 4
NG = NCHUNK // NBUF


def _chan_offset(v: int) -> jax.Array:
    base = v * LANES
    c0, c1 = base // SEQ, (base + LANES - 1) // SEQ
    if c0 == c1:
        return jnp.full((LANES,), c0 * VOCAB, jnp.int32)
    k = c1 * SEQ - base
    return jnp.where(lax.iota(jnp.int32, LANES) < k,
                     jnp.int32(c0 * VOCAB), jnp.int32(c1 * VOCAB))


def _sc_body(x_hbm, tab_hbm, out_hbm, tab_v, bufs, sin, sout):
    wid = lax.axis_index("s") * NC + lax.axis_index("c")
    pltpu.sync_copy(tab_hbm, tab_v)
    base = wid * ROWS_PER_TILE * ROW

    def in_copy(k, b):
        off = pl.multiple_of(base + k * CW, 8)
        return pltpu.make_async_copy(x_hbm.at[pl.ds(off, CW)], bufs[b], sin[b])

    def out_copy(k, b):
        off = pl.multiple_of(base + k * CW, 8)
        return pltpu.make_async_copy(bufs[b], out_hbm.at[pl.ds(off, CW)],
                                     sout[b])

    def gather_chunk(b):
        def row_body(r, c):
            ro = r * ROW
            for v in range(VPR):
                o = ro + v * LANES
                idx = (bufs[b][pl.ds(o, LANES)].astype(jnp.int32)
                       + _chan_offset(v))
                bufs[b][pl.ds(o, LANES)] = plsc.load_gather(tab_v, [idx])
            return c
        lax.fori_loop(0, NB, row_body, 0, unroll=False)

    for b in range(NBUF - 1):
        in_copy(b, b).start()

    def g_body(g, carry):
        for b in range(NBUF):
            k = g * NBUF + b
            bp = (b - 1) % NBUF
            if b == 0:
                @pl.when(g == 0)
                def _():
                    in_copy(NBUF - 1, NBUF - 1).start()

                @pl.when(g > 0)
                def _():
                    out_copy(0, bp).wait()
                    in_copy(k + NBUF - 1, bp).start()
            else:
                out_copy(0, bp).wait()

                @pl.when(g < NG - 1)
                def _():
                    in_copy(k + NBUF - 1, bp).start()
            in_copy(k, b).wait()
            gather_chunk(b)
            out_copy(k, b).start()
        return carry

    lax.fori_loop(0, NG, g_body, 0, unroll=False)
    out_copy(NCHUNK - 1, (NCHUNK - 1) % NBUF).wait()


@functools.partial(jax.jit, static_argnames=())
def _run(x_flat, tab):
    mesh = plsc.VectorSubcoreMesh(core_axis_name="c", subcore_axis_name="s")
    return pl.kernel(
        lambda x, t, o, tv, b0, b1, b2, b3, si0, si1, si2, si3, so0, so1,
               so2, so3: _sc_body(x, t, o, tv, (b0, b1, b2, b3),
                                  (si0, si1, si2, si3), (so0, so1, so2, so3)),
        out_type=jax.ShapeDtypeStruct((BS * ROW,), jnp.float32),
        mesh=mesh,
        scratch_types=[pltpu.VMEM((NCAT * VOCAB,), jnp.float32)]
        + [pltpu.VMEM((CW,), jnp.float32)] * NBUF
        + [pltpu.SemaphoreType.DMA] * (2 * NBUF),
        compiler_params=pltpu.CompilerParams(needs_layout_passes=False),
    )(x_flat, tab)


def kernel(x, W0, W1, W2, W3, W4, W5, W6, W7, W8, W9):
    tab = jnp.concatenate(
        [W0, W1, W2, W3, W4, W5, W6, W7, W8, W9], axis=0
    ).reshape(NCAT * VOCAB)
    out = _run(x.reshape(BS * ROW), tab)
    return out.reshape(BS, NV, SEQ)


# speedup vs baseline: 184.7989x; 1.4597x over previous
import functools

import jax
import jax.numpy as jnp
from jax import lax
from jax.experimental import pallas as pl
from jax.experimental.pallas import tpu as pltpu
from jax.experimental.pallas import tpu_sc as plsc

BS, NV, SEQ = 4096, 26, 200
NCAT = 10
VOCAB = 1000
LANES = 16
NC, NS = 2, 16
NWORKERS = NC * NS
ROWS_PER_TILE = BS // NWORKERS
NB = 2
NCHUNK = ROWS_PER_TILE // NB
NBUF = 4
NG = NCHUNK // NBUF


def _sc_body(x_hbm, tab_hbm, out_hbm, tab_v, bufs, sin, sout):
    wid = lax.axis_index("s") * NC + lax.axis_index("c")
    pltpu.sync_copy(tab_hbm, tab_v)
    row0 = wid * ROWS_PER_TILE

    def in_copy(k, b):
        return pltpu.make_async_copy(
            x_hbm.at[pl.ds(row0 + k * NB, NB)], bufs[b], sin[b])

    def out_copy(k, b):
        return pltpu.make_async_copy(
            bufs[b], out_hbm.at[pl.ds(row0 + k * NB, NB)], sout[b])

    def gather_chunk(b):
        buf = bufs[b]

        def row_body(r, carry):
            for ch in range(NCAT):
                off = jnp.int32(ch * VOCAB)

                def gath(v):
                    return plsc.load_gather(
                        tab_v, [v.astype(jnp.int32) + off])

                for s0 in range(0, SEQ - 2 * LANES, LANES):
                    sl = (r, ch, pl.ds(s0, LANES))
                    buf[sl] = gath(buf[sl])
                sa = (r, ch, pl.ds(SEQ - 24, LANES))
                sb = (r, ch, pl.ds(SEQ - LANES, LANES))
                va, vb = buf[sa], buf[sb]
                buf[sa] = gath(va)
                buf[sb] = gath(vb)
            return carry

        lax.fori_loop(0, NB, row_body, 0, unroll=False)

    for b in range(NBUF - 1):
        in_copy(b, b).start()

    def g_body(g, carry):
        for b in range(NBUF):
            k = g * NBUF + b
            bp = (b - 1) % NBUF
            if b == 0:
                @pl.when(g == 0)
                def _():
                    in_copy(NBUF - 1, NBUF - 1).start()

                @pl.when(g > 0)
                def _():
                    out_copy(0, bp).wait()
                    in_copy(k + NBUF - 1, bp).start()
            else:
                out_copy(0, bp).wait()

                @pl.when(g < NG - 1)
                def _():
                    in_copy(k + NBUF - 1, bp).start()
            in_copy(k, b).wait()
            gather_chunk(b)
            out_copy(k, b).start()
        return carry

    lax.fori_loop(0, NG, g_body, 0, unroll=False)
    out_copy(NCHUNK - 1, (NCHUNK - 1) % NBUF).wait()


@functools.partial(jax.jit, static_argnames=())
def _run(x, tab):
    mesh = plsc.VectorSubcoreMesh(core_axis_name="c", subcore_axis_name="s")
    return pl.kernel(
        lambda x_, t, o, tv, b0, b1, b2, b3, si0, si1, si2, si3, so0, so1,
               so2, so3: _sc_body(x_, t, o, tv, (b0, b1, b2, b3),
                                  (si0, si1, si2, si3), (so0, so1, so2, so3)),
        out_type=jax.ShapeDtypeStruct((BS, NV, SEQ), jnp.float32),
        mesh=mesh,
        scratch_types=[pltpu.VMEM((NCAT * VOCAB,), jnp.float32)]
        + [pltpu.VMEM((NB, NV, SEQ), jnp.float32)] * NBUF
        + [pltpu.SemaphoreType.DMA] * (2 * NBUF),
        compiler_params=pltpu.CompilerParams(
            needs_layout_passes=False, use_tc_tiling_on_sc=True),
    )(x, tab)


def kernel(x, W0, W1, W2, W3, W4, W5, W6, W7, W8, W9):
    tab = jnp.concatenate(
        [W0, W1, W2, W3, W4, W5, W6, W7, W8, W9], axis=0
    ).reshape(NCAT * VOCAB)
    return _run(x, tab)
